# baseline (speedup 1.0000x reference)
import jax
import jax.numpy as jnp
from jax.experimental import pallas as pl
from jax.experimental.pallas import tpu as pltpu

_BM = 256
_BK = 2048


def _gat_kernel(adj_ref, x_ref, w_ref, out_ref, acc_ref):
    k = pl.program_id(1)

    @pl.when(k == 0)
    def _():
        acc_ref[...] = jnp.zeros_like(acc_ref)

    acc_ref[...] += jnp.dot(
        adj_ref[...], x_ref[...], preferred_element_type=jnp.float32
    )

    @pl.when(k == pl.num_programs(1) - 1)
    def _():
        out_ref[...] = jnp.dot(
            acc_ref[...], w_ref[...], preferred_element_type=jnp.float32
        )


def kernel(x, adj, W):
    n, d = x.shape
    grid = (n // _BM, n // _BK)
    return pl.pallas_call(
        _gat_kernel,
        grid=grid,
        in_specs=[
            pl.BlockSpec((_BM, _BK), lambda i, k: (i, k)),
            pl.BlockSpec((_BK, d), lambda i, k: (k, 0)),
            pl.BlockSpec((d, d), lambda i, k: (0, 0)),
        ],
        out_specs=pl.BlockSpec((_BM, d), lambda i, k: (i, 0)),
        out_shape=jax.ShapeDtypeStruct((n, d), jnp.float32),
        scratch_shapes=[pltpu.VMEM((_BM, d), jnp.float32)],
        compiler_params=pltpu.CompilerParams(
            dimension_semantics=("parallel", "arbitrary"),
        ),
    )(adj, x, W)


# speedup vs baseline: 1.1406x; 1.1406x over previous
import jax
import jax.numpy as jnp
from jax.experimental import pallas as pl
from jax.experimental.pallas import tpu as pltpu

_BM = 256
_BK = 2048


def _gat_kernel(adj_ref, x_ref, w_ref, out_ref, acc_ref, xbf_ref):
    i = pl.program_id(0)
    k = pl.program_id(1)

    @pl.when((i == 0) & (k == 0))
    def _():
        xbf_ref[...] = x_ref[...].astype(jnp.bfloat16)

    @pl.when(k == 0)
    def _():
        acc_ref[...] = jnp.zeros_like(acc_ref)

    adj_bf = adj_ref[...].astype(jnp.bfloat16)
    x_bf = xbf_ref[pl.ds(k * _BK, _BK), :]
    acc_ref[...] += jnp.dot(adj_bf, x_bf, preferred_element_type=jnp.float32)

    @pl.when(k == pl.num_programs(1) - 1)
    def _():
        out_ref[...] = jnp.dot(
            acc_ref[...], w_ref[...], preferred_element_type=jnp.float32
        )


def kernel(x, adj, W):
    n, d = x.shape
    grid = (n // _BM, n // _BK)
    return pl.pallas_call(
        _gat_kernel,
        grid=grid,
        in_specs=[
            pl.BlockSpec((_BM, _BK), lambda i, k: (i, k)),
            pl.BlockSpec((n, d), lambda i, k: (0, 0)),
            pl.BlockSpec((d, d), lambda i, k: (0, 0)),
        ],
        out_specs=pl.BlockSpec((_BM, d), lambda i, k: (i, 0)),
        out_shape=jax.ShapeDtypeStruct((n, d), jnp.float32),
        scratch_shapes=[
            pltpu.VMEM((_BM, d), jnp.float32),
            pltpu.VMEM((n, d), jnp.bfloat16),
        ],
        compiler_params=pltpu.CompilerParams(
            dimension_semantics=("parallel", "arbitrary"),
        ),
    )(adj, x, W)
